# baseline (speedup 1.0000x reference)
import functools

import jax
import jax.numpy as jnp
from jax import lax
from jax.experimental import pallas as pl
from jax.experimental.pallas import tpu as pltpu
from jax.experimental.pallas import tpu_sc as plsc

N = 10000
E = 320000
E1 = E + N
F = 128
H = 8
C = 16
G = 64

NP = 10240
NW = 32
K = 128
KS = 128
M = 84
EPW = K * M
E_PAD = NW * EPW
ROWS_PER_SUB = NP // 16

_P = lax.Precision.HIGHEST

_GDN = lax.GatherDimensionNumbers(
    offset_dims=(), collapsed_slice_dims=(0,), start_index_map=(0,))


def _splat16(vec, idx):
    return lax.gather(vec, idx[:, None], _GDN, slice_sizes=(1,),
                      mode=lax.GatherScatterMode.PROMISE_IN_BOUNDS)


def _tc_layer_body(first, *refs):
    if first:
        (p0_r, b_r, w_r, as_r, ad_r, hlo_r, hhi_r, ts_r, td_r) = refs
        a = p0_r[...]
    else:
        (plo0, phi0, plo1, phi1, b_r, w_r, as_r, ad_r,
         hlo_r, hhi_r, ts_r, td_r) = refs
        a = jnp.concatenate([plo0[...] + plo1[...], phi0[...] + phi1[...]],
                            axis=1)
        a = jnp.maximum(a + b_r[...], 0.0)
    h = jnp.dot(a, w_r[...], precision=lax.Precision.DEFAULT)
    hlo_r[...] = h[:, :F // 2]
    hhi_r[...] = h[:, F // 2:]
    ts_r[...] = jnp.dot(h, as_r[...], precision=_P)
    td_r[...] = jnp.dot(h, ad_r[...], precision=_P)


def _tc_layer(prev, b, w, a_sd, a_dd, first):
    BN = 1024
    grid = (NP // BN,)
    row_spec = pl.BlockSpec((BN, F), lambda i: (i, 0))
    half_spec = pl.BlockSpec((BN, F // 2), lambda i: (i, 0))
    logit_spec = pl.BlockSpec((BN, 2 * H), lambda i: (i, 0))
    full = lambda s: pl.BlockSpec(s, lambda i: (0, 0))
    in_specs = ([row_spec] if first else [half_spec] * 4) + [
        full((1, F)), full((F, F)), full((F, 2 * H)), full((F, 2 * H))]
    return pl.pallas_call(
        functools.partial(_tc_layer_body, first),
        grid=grid,
        in_specs=in_specs,
        out_specs=[half_spec, half_spec, logit_spec, logit_spec],
        out_shape=[jax.ShapeDtypeStruct((NP, F // 2), jnp.float32),
                   jax.ShapeDtypeStruct((NP, F // 2), jnp.float32),
                   jax.ShapeDtypeStruct((NP, 2 * H), jnp.float32),
                   jax.ShapeDtypeStruct((NP, 2 * H), jnp.float32)],
    )(*prev, b, w, a_sd, a_dd)


def _rden_body(a_r, b_r, r_r):
    r_r[...] = 1.0 / (a_r[...] + b_r[...] + 1e-16)


def _rden(dena, denb):
    full = lambda s: pl.BlockSpec(s, lambda: (0, 0))
    return pl.pallas_call(
        _rden_body,
        in_specs=[full((NP, 2 * H)), full((NP, 2 * H))],
        out_specs=full((NP, 2 * H)),
        out_shape=jax.ShapeDtypeStruct((NP, 2 * H), jnp.float32),
    )(dena, denb)


def _sc_a_body(ts_hbm, td_hbm, src2_hbm, dst2_hbm, z16_hbm,
               ex_hbm, dena_hbm, denb_hbm,
               idxs, idxd, rows_s, rows_d, exb, den_sh,
               sg0, sg1, sg2, sg3):
    cid = lax.axis_index("c")
    sid = lax.axis_index("s")
    wid = sid * 2 + cid
    sgs = (sg0, sg1, sg2, sg3)

    pltpu.sync_copy(z16_hbm.at[pl.ds(sid * ROWS_PER_SUB, ROWS_PER_SUB)],
                    den_sh.at[pl.ds(sid * ROWS_PER_SUB, ROWS_PER_SUB)])
    pltpu.sync_copy(src2_hbm.at[pl.ds(wid * M, M)], idxs)
    pltpu.sync_copy(dst2_hbm.at[pl.ds(wid * M, M)], idxd)
    plsc.subcore_barrier()

    def fire_g(m, b):
        pltpu.async_copy(ts_hbm.at[idxs.at[m]], rows_s.at[b], sgs[b])
        pltpu.async_copy(td_hbm.at[idxd.at[m]], rows_d.at[b], sgs[b])

    def wait_g(m, b):
        pltpu.make_async_copy(ts_hbm.at[idxs.at[m]], rows_s.at[b],
                              sgs[b]).wait()
        pltpu.make_async_copy(td_hbm.at[idxd.at[m]], rows_d.at[b],
                              sgs[b]).wait()

    def proc(m, b):
        base = wid * EPW + m * K

        @plsc.parallel_loop(0, K, unroll=8)
        def _(k):
            e = rows_s[b, k, :] + rows_d[b, k, :]
            exb[k, :] = jnp.exp(jnp.maximum(e, 0.2 * e))

        pltpu.sync_copy(exb, ex_hbm.at[pl.ds(base, K)])
        pltpu.sync_copy(exb, den_sh.at[idxd.at[m]], add=True)

    for b in range(4):
        fire_g(b, b)

    def step(t, _):
        for b in range(4):
            m = 4 * t + b
            wait_g(m, b)
            proc(m, b)

            @pl.when(t < M // 4 - 1)
            def _():
                fire_g(m + 4, b)
        return 0

    lax.fori_loop(0, M // 4, step, 0)
    plsc.subcore_barrier()

    sl = pl.ds(sid * ROWS_PER_SUB, ROWS_PER_SUB)

    @pl.when(cid == 0)
    def _():
        pltpu.sync_copy(den_sh.at[sl], dena_hbm.at[sl])

    @pl.when(cid == 1)
    def _():
        pltpu.sync_copy(den_sh.at[sl], denb_hbm.at[sl])


def _sc_a(ts, td, src2, dst2, z16):
    mesh = plsc.VectorSubcoreMesh(core_axis_name="c", subcore_axis_name="s")
    f = pl.kernel(
        _sc_a_body,
        out_type=(jax.ShapeDtypeStruct((E_PAD, 2 * H), jnp.float32),
                  jax.ShapeDtypeStruct((NP, 2 * H), jnp.float32),
                  jax.ShapeDtypeStruct((NP, 2 * H), jnp.float32)),
        mesh=mesh,
        scratch_types=[
            pltpu.VMEM((M, KS), jnp.int32),
            pltpu.VMEM((M, KS), jnp.int32),
            pltpu.VMEM((4, K, 2 * H), jnp.float32),
            pltpu.VMEM((4, K, 2 * H), jnp.float32),
            pltpu.VMEM((K, 2 * H), jnp.float32),
            pltpu.VMEM_SHARED((NP, 2 * H), jnp.float32),
            pltpu.SemaphoreType.DMA,
            pltpu.SemaphoreType.DMA,
            pltpu.SemaphoreType.DMA,
            pltpu.SemaphoreType.DMA,
        ],
        compiler_params=pltpu.CompilerParams(use_tc_tiling_on_sc=False),
    )
    return f(ts, td, src2, dst2, z16)


def _sc_b_body(hlo_hbm, hhi_hbm, ex_hbm, rden_hbm, src2_hbm, dst2_hbm,
               z64_hbm, oalo_hbm, oahi_hbm, oblo_hbm, obhi_hbm,
               idxs, idxd, rowsh, msgb, exb, rdb, acc_sh,
               sg0, sg1, sg2, sg3, ss0, ss1):
    cid = lax.axis_index("c")
    sid = lax.axis_index("s")
    wid = sid * 2 + cid
    sgs = (sg0, sg1, sg2, sg3)
    sss = (ss0, ss1)
    sl = pl.ds(sid * ROWS_PER_SUB, ROWS_PER_SUB)

    pltpu.sync_copy(src2_hbm.at[pl.ds(wid * M, M)], idxs)
    pltpu.sync_copy(dst2_hbm.at[pl.ds(wid * M, M)], idxd)

    for half, (h_hbm, oa_hbm, ob_hbm) in enumerate(
            [(hlo_hbm, oalo_hbm, oblo_hbm), (hhi_hbm, oahi_hbm, obhi_hbm)]):
        pltpu.sync_copy(z64_hbm.at[sl], acc_sh.at[sl])
        plsc.subcore_barrier()

        def fire_g(m, b):
            base = wid * EPW + m * K
            pltpu.async_copy(h_hbm.at[idxs.at[m]], rowsh.at[b], sgs[b])
            pltpu.async_copy(ex_hbm.at[pl.ds(base, K)], exb.at[b], sgs[b])
            pltpu.async_copy(rden_hbm.at[idxd.at[m]], rdb.at[b], sgs[b])

        def wait_g(m, b):
            base = wid * EPW + m * K
            pltpu.make_async_copy(h_hbm.at[idxs.at[m]], rowsh.at[b],
                                  sgs[b]).wait()
            pltpu.make_async_copy(ex_hbm.at[pl.ds(base, K)], exb.at[b],
                                  sgs[b]).wait()
            pltpu.make_async_copy(rden_hbm.at[idxd.at[m]], rdb.at[b],
                                  sgs[b]).wait()

        def proc(m, b):
            @plsc.parallel_loop(0, K, unroll=4)
            def _(k):
                al = exb[b, k, :] * rdb[b, k, :]
                for j in range(H // 2):
                    spl = _splat16(al, jnp.full((16,), half * (H // 2) + j,
                                                jnp.int32))
                    s = pl.ds(j * C, C)
                    msgb[b % 2, k, s] = rowsh[b, k, s] * spl

            pltpu.async_copy(msgb.at[b % 2], acc_sh.at[idxd.at[m]],
                             sss[b % 2], add=True)

        def wait_s(m, p):
            pltpu.make_async_copy(msgb.at[p], acc_sh.at[idxd.at[m]],
                                  sss[p]).wait()

        for b in range(4):
            fire_g(b, b)

        def step(t, _):
            for b in range(4):
                m = 4 * t + b
                wait_g(m, b)

                @pl.when(m >= 2)
                def _():
                    wait_s(m - 2, b % 2)

                proc(m, b)

                @pl.when(t < M // 4 - 1)
                def _():
                    fire_g(m + 4, b)
            return 0

        lax.fori_loop(0, M // 4, step, 0)
        wait_s(M - 2, 0)
        wait_s(M - 1, 1)
        plsc.subcore_barrier()

        @pl.when(cid == 0)
        def _():
            pltpu.sync_copy(acc_sh.at[sl], oa_hbm.at[sl])

        @pl.when(cid == 1)
        def _():
            pltpu.sync_copy(acc_sh.at[sl], ob_hbm.at[sl])

        plsc.subcore_barrier()


def _sc_b(hlo, hhi, ex, rden, src2, dst2, z64):
    mesh = plsc.VectorSubcoreMesh(core_axis_name="c", subcore_axis_name="s")
    f = pl.kernel(
        _sc_b_body,
        out_type=(jax.ShapeDtypeStruct((NP, F // 2), jnp.float32),
                  jax.ShapeDtypeStruct((NP, F // 2), jnp.float32),
                  jax.ShapeDtypeStruct((NP, F // 2), jnp.float32),
                  jax.ShapeDtypeStruct((NP, F // 2), jnp.float32)),
        mesh=mesh,
        scratch_types=[
            pltpu.VMEM((M, KS), jnp.int32),
            pltpu.VMEM((M, KS), jnp.int32),
            pltpu.VMEM((4, K, F // 2), jnp.float32),
            pltpu.VMEM((2, K, F // 2), jnp.float32),
            pltpu.VMEM((4, K, 2 * H), jnp.float32),
            pltpu.VMEM((4, K, 2 * H), jnp.float32),
            pltpu.VMEM_SHARED((NP, F // 2), jnp.float32),
            pltpu.SemaphoreType.DMA,
            pltpu.SemaphoreType.DMA,
            pltpu.SemaphoreType.DMA,
            pltpu.SemaphoreType.DMA,
            pltpu.SemaphoreType.DMA,
            pltpu.SemaphoreType.DMA,
        ],
        compiler_params=pltpu.CompilerParams(use_tc_tiling_on_sc=False),
    )
    return f(hlo, hhi, ex, rden, src2, dst2, z64)


def _pool_body(plo0, phi0, plo1, phi1, b_r, batch_r, fw1_r, fb1_r, fw2_r,
               fb2_r, y_r):
    h3 = jnp.concatenate([plo0[...] + plo1[...], phi0[...] + phi1[...]],
                         axis=1)
    h3 = jnp.maximum(h3 + b_r[...], 0.0)
    rows = lax.broadcasted_iota(jnp.int32, (NP, 1), 0)
    h3 = jnp.where(rows < N, h3, 0.0)
    gids = lax.broadcasted_iota(jnp.int32, (1, G), 1)
    onehot = (batch_r[...] == gids).astype(jnp.float32)
    sums = lax.dot_general(onehot, h3, (((0,), (0,)), ((), ())),
                           precision=_P)
    cnt = jnp.sum(onehot, axis=0)[:, None]
    g = sums / jnp.maximum(cnt, 1.0)
    dp = lax.Precision.DEFAULT
    z = jnp.maximum(jnp.dot(g, fw1_r[...], precision=dp) + fb1_r[...], 0.0)
    y_r[...] = jnp.dot(z, fw2_r[...], precision=dp) + fb2_r[...]


def _pool(prev, b3, batchp, fw1, fb1, fw2, fb2):
    full = lambda s: pl.BlockSpec(s, lambda: (0, 0))
    return pl.pallas_call(
        _pool_body,
        in_specs=[full((NP, F // 2))] * 4 + [full((1, F)),
                  full((NP, 1)), full((F, 10)), full((1, 10)),
                  full((10, 1)), full((1, 1))],
        out_specs=full((G, 1)),
        out_shape=jax.ShapeDtypeStruct((G, 1), jnp.float32),
    )(*prev, b3, batchp, fw1, fb1, fw2, fb2)


def _dup_table(a):
    blk = a[:, :, None] * jnp.eye(H, dtype=a.dtype)[:, None, :]
    A = blk.reshape(F, H)
    return jnp.concatenate([A, A], axis=1)


def kernel(x, edge_index, batch, W1, a_s1, a_d1, b1, W2, a_s2, a_d2, b2,
           W3, a_s3, a_d3, b3, fw1, fb1, fw2, fb2):
    i32 = jnp.int32
    loop = jnp.arange(N, dtype=i32)
    pad = E_PAD - E1
    src = jnp.concatenate([edge_index[0].astype(i32), loop,
                           jnp.zeros((pad,), i32)])
    dst = jnp.concatenate([edge_index[1].astype(i32), loop,
                           jnp.full((pad,), N, i32)])
    src2 = src.reshape(E_PAD // KS, KS)
    dst2 = dst.reshape(E_PAD // KS, KS)

    xp = jnp.concatenate([x, jnp.zeros((NP - N, F), jnp.float32)])
    z16 = jnp.zeros((NP, 2 * H), jnp.float32)
    z64 = jnp.zeros((NP, F // 2), jnp.float32)
    batchp = jnp.concatenate([batch.astype(i32),
                              jnp.full((NP - N,), G, i32)]).reshape(NP, 1)

    prev = (xp,)
    bias_prev = jnp.zeros((1, F), jnp.float32)
    for li, (W, a_s, a_d, b) in enumerate(
            [(W1, a_s1, a_d1, b1), (W2, a_s2, a_d2, b2),
             (W3, a_s3, a_d3, b3)]):
        hlo, hhi, ts, td = _tc_layer(prev, bias_prev, W,
                                     _dup_table(a_s), _dup_table(a_d),
                                     li == 0)
        ex, dena, denb = _sc_a(ts, td, src2, dst2, z16)
        rden = _rden(dena, denb)
        prev = _sc_b(hlo, hhi, ex, rden, src2, dst2, z64)
        bias_prev = b.reshape(1, F)

    return _pool(prev, bias_prev, batchp, fw1, fb1.reshape(1, 10),
                 fw2, fb2.reshape(1, 1))


# speedup vs baseline: 1.6970x; 1.6970x over previous
import functools

import jax
import jax.numpy as jnp
from jax import lax
from jax.experimental import pallas as pl
from jax.experimental.pallas import tpu as pltpu
from jax.experimental.pallas import tpu_sc as plsc

N = 10000
E = 320000
E1 = E + N
F = 128
H = 8
C = 16
G = 64

NP = 10240
NW = 32
K = 128
KS = 128
M = 82
EPW = K * M
E_PAD = NW * EPW
ROWS_PER_SUB = NP // 16

_P = lax.Precision.HIGHEST

_GDN = lax.GatherDimensionNumbers(
    offset_dims=(), collapsed_slice_dims=(0,), start_index_map=(0,))


def _splat16(vec, idx):
    return lax.gather(vec, idx[:, None], _GDN, slice_sizes=(1,),
                      mode=lax.GatherScatterMode.PROMISE_IN_BOUNDS)


def _tc_layer_body(first, *refs):
    if first:
        (p0_r, b_r, w_r, as_r, ad_r, hlo_r, hhi_r, ts_r, td_r) = refs
        a = p0_r[...]
    else:
        (plo0, phi0, plo1, phi1, b_r, w_r, as_r, ad_r,
         hlo_r, hhi_r, ts_r, td_r) = refs
        a = jnp.concatenate([plo0[...] + plo1[...], phi0[...] + phi1[...]],
                            axis=1)
        a = jnp.maximum(a + b_r[...], 0.0)
    h = jnp.dot(a, w_r[...], precision=lax.Precision.DEFAULT)
    hlo_r[...] = h[:, :F // 2]
    hhi_r[...] = h[:, F // 2:]
    ts_r[...] = jnp.dot(h, as_r[...], precision=_P)
    td_r[...] = jnp.dot(h, ad_r[...], precision=_P)


def _tc_layer(prev, b, w, a_sd, a_dd, first):
    BN = 1024
    grid = (NP // BN,)
    row_spec = pl.BlockSpec((BN, F), lambda i: (i, 0))
    half_spec = pl.BlockSpec((BN, F // 2), lambda i: (i, 0))
    logit_spec = pl.BlockSpec((BN, 2 * H), lambda i: (i, 0))
    full = lambda s: pl.BlockSpec(s, lambda i: (0, 0))
    in_specs = ([row_spec] if first else [half_spec] * 4) + [
        full((1, F)), full((F, F)), full((F, 2 * H)), full((F, 2 * H))]
    return pl.pallas_call(
        functools.partial(_tc_layer_body, first),
        grid=grid,
        in_specs=in_specs,
        out_specs=[half_spec, half_spec, logit_spec, logit_spec],
        out_shape=[jax.ShapeDtypeStruct((NP, F // 2), jnp.float32),
                   jax.ShapeDtypeStruct((NP, F // 2), jnp.float32),
                   jax.ShapeDtypeStruct((NP, 2 * H), jnp.float32),
                   jax.ShapeDtypeStruct((NP, 2 * H), jnp.float32)],
    )(*prev, b, w, a_sd, a_dd)


def _rden_body(a_r, b_r, r_r):
    r_r[...] = 1.0 / (a_r[...] + b_r[...] + 1e-16)


def _rden(dena, denb):
    full = lambda s: pl.BlockSpec(s, lambda: (0, 0))
    return pl.pallas_call(
        _rden_body,
        in_specs=[full((NP, 2 * H)), full((NP, 2 * H))],
        out_specs=full((NP, 2 * H)),
        out_shape=jax.ShapeDtypeStruct((NP, 2 * H), jnp.float32),
    )(dena, denb)


def _sc_a_body(ts_hbm, td_hbm, src2_hbm, dst2_hbm, z16_hbm,
               ex_hbm, dena_hbm, denb_hbm,
               idxs, idxd, rows_s, rows_d, exb, den_sh,
               sg0, sg1, sg2, sg3):
    cid = lax.axis_index("c")
    sid = lax.axis_index("s")
    wid = sid * 2 + cid
    sgs = (sg0, sg1, sg2, sg3)

    pltpu.sync_copy(z16_hbm.at[pl.ds(sid * ROWS_PER_SUB, ROWS_PER_SUB)],
                    den_sh.at[pl.ds(sid * ROWS_PER_SUB, ROWS_PER_SUB)])
    pltpu.sync_copy(src2_hbm.at[pl.ds(wid * M, M)], idxs)
    pltpu.sync_copy(dst2_hbm.at[pl.ds(wid * M, M)], idxd)
    plsc.subcore_barrier()

    def fire_g(m, b):
        pltpu.async_copy(ts_hbm.at[idxs.at[m]], rows_s.at[b], sgs[b])
        pltpu.async_copy(td_hbm.at[idxd.at[m]], rows_d.at[b], sgs[b])

    def wait_g(m, b):
        pltpu.make_async_copy(ts_hbm.at[idxs.at[m]], rows_s.at[b],
                              sgs[b]).wait()
        pltpu.make_async_copy(td_hbm.at[idxd.at[m]], rows_d.at[b],
                              sgs[b]).wait()

    def proc(m, b):
        base = wid * EPW + m * K

        @plsc.parallel_loop(0, K, unroll=8)
        def _(k):
            e = rows_s[b, k, :] + rows_d[b, k, :]
            exb[k, :] = jnp.exp(jnp.maximum(e, 0.2 * e))

        pltpu.sync_copy(exb, ex_hbm.at[pl.ds(base, K)])
        pltpu.sync_copy(exb, den_sh.at[idxd.at[m]], add=True)

    fire_g(0, 0)

    def step(t, _):
        m0 = 2 * t
        m1 = m0 + 1
        fire_g(m1, 1)
        wait_g(m0, 0)
        proc(m0, 0)

        @pl.when(t < M // 2 - 1)
        def _():
            fire_g(m0 + 2, 0)

        wait_g(m1, 1)
        proc(m1, 1)
        return 0

    lax.fori_loop(0, M // 2, step, 0)
    plsc.subcore_barrier()

    sl = pl.ds(sid * ROWS_PER_SUB, ROWS_PER_SUB)

    @pl.when(cid == 0)
    def _():
        pltpu.sync_copy(den_sh.at[sl], dena_hbm.at[sl])

    @pl.when(cid == 1)
    def _():
        pltpu.sync_copy(den_sh.at[sl], denb_hbm.at[sl])


def _sc_a(ts, td, src2, dst2, z16):
    mesh = plsc.VectorSubcoreMesh(core_axis_name="c", subcore_axis_name="s")
    f = pl.kernel(
        _sc_a_body,
        out_type=(jax.ShapeDtypeStruct((E_PAD, 2 * H), jnp.float32),
                  jax.ShapeDtypeStruct((NP, 2 * H), jnp.float32),
                  jax.ShapeDtypeStruct((NP, 2 * H), jnp.float32)),
        mesh=mesh,
        scratch_types=[
            pltpu.VMEM((M, KS), jnp.int32),
            pltpu.VMEM((M, KS), jnp.int32),
            pltpu.VMEM((2, K, 2 * H), jnp.float32),
            pltpu.VMEM((2, K, 2 * H), jnp.float32),
            pltpu.VMEM((K, 2 * H), jnp.float32),
            pltpu.VMEM_SHARED((NP, 2 * H), jnp.float32),
            pltpu.SemaphoreType.DMA,
            pltpu.SemaphoreType.DMA,
            pltpu.SemaphoreType.DMA,
            pltpu.SemaphoreType.DMA,
        ],
        compiler_params=pltpu.CompilerParams(use_tc_tiling_on_sc=False),
    )
    return f(ts, td, src2, dst2, z16)


def _sc_b_body(hlo_hbm, hhi_hbm, ex_hbm, rden_hbm, src2_hbm, dst2_hbm,
               z64_hbm, oalo_hbm, oahi_hbm, oblo_hbm, obhi_hbm,
               idxs, idxd, rowsh, msgb, exb, rdb, acc_sh,
               sg0, sg1, sg2, sg3, ss0, ss1):
    cid = lax.axis_index("c")
    sid = lax.axis_index("s")
    wid = sid * 2 + cid
    sgs = (sg0, sg1, sg2, sg3)
    sss = (ss0, ss1)
    sl = pl.ds(sid * ROWS_PER_SUB, ROWS_PER_SUB)

    pltpu.sync_copy(src2_hbm.at[pl.ds(wid * M, M)], idxs)
    pltpu.sync_copy(dst2_hbm.at[pl.ds(wid * M, M)], idxd)

    for half, (h_hbm, oa_hbm, ob_hbm) in enumerate(
            [(hlo_hbm, oalo_hbm, oblo_hbm), (hhi_hbm, oahi_hbm, obhi_hbm)]):
        pltpu.sync_copy(z64_hbm.at[sl], acc_sh.at[sl])
        plsc.subcore_barrier()

        def fire_g(m, b):
            base = wid * EPW + m * K
            pltpu.async_copy(h_hbm.at[idxs.at[m]], rowsh.at[b], sgs[b])
            pltpu.async_copy(ex_hbm.at[pl.ds(base, K)], exb.at[b], sgs[b])
            pltpu.async_copy(rden_hbm.at[idxd.at[m]], rdb.at[b], sgs[b])

        def wait_g(m, b):
            base = wid * EPW + m * K
            pltpu.make_async_copy(h_hbm.at[idxs.at[m]], rowsh.at[b],
                                  sgs[b]).wait()
            pltpu.make_async_copy(ex_hbm.at[pl.ds(base, K)], exb.at[b],
                                  sgs[b]).wait()
            pltpu.make_async_copy(rden_hbm.at[idxd.at[m]], rdb.at[b],
                                  sgs[b]).wait()

        def proc(m, b):
            @plsc.parallel_loop(0, K, unroll=4)
            def _(k):
                al = exb[b, k, :] * rdb[b, k, :]
                for j in range(H // 2):
                    spl = _splat16(al, jnp.full((16,), half * (H // 2) + j,
                                                jnp.int32))
                    s = pl.ds(j * C, C)
                    msgb[b % 2, k, s] = rowsh[b, k, s] * spl

            pltpu.async_copy(msgb.at[b % 2], acc_sh.at[idxd.at[m]],
                             sss[b % 2], add=True)

        def wait_s(m, p):
            pltpu.make_async_copy(msgb.at[p], acc_sh.at[idxd.at[m]],
                                  sss[p]).wait()

        fire_g(0, 0)

        def step(t, _):
            m0 = 2 * t
            m1 = m0 + 1
            fire_g(m1, 1)
            wait_g(m0, 0)

            @pl.when(t >= 1)
            def _():
                wait_s(m0 - 2, 0)

            proc(m0, 0)

            @pl.when(t < M // 2 - 1)
            def _():
                fire_g(m0 + 2, 0)

            wait_g(m1, 1)

            @pl.when(t >= 1)
            def _():
                wait_s(m1 - 2, 1)

            proc(m1, 1)
            return 0

        lax.fori_loop(0, M // 2, step, 0)
        wait_s(M - 2, 0)
        wait_s(M - 1, 1)
        plsc.subcore_barrier()

        @pl.when(cid == 0)
        def _():
            pltpu.sync_copy(acc_sh.at[sl], oa_hbm.at[sl])

        @pl.when(cid == 1)
        def _():
            pltpu.sync_copy(acc_sh.at[sl], ob_hbm.at[sl])

        plsc.subcore_barrier()


def _sc_b(hlo, hhi, ex, rden, src2, dst2, z64):
    mesh = plsc.VectorSubcoreMesh(core_axis_name="c", subcore_axis_name="s")
    f = pl.kernel(
        _sc_b_body,
        out_type=(jax.ShapeDtypeStruct((NP, F // 2), jnp.float32),
                  jax.ShapeDtypeStruct((NP, F // 2), jnp.float32),
                  jax.ShapeDtypeStruct((NP, F // 2), jnp.float32),
                  jax.ShapeDtypeStruct((NP, F // 2), jnp.float32)),
        mesh=mesh,
        scratch_types=[
            pltpu.VMEM((M, KS), jnp.int32),
            pltpu.VMEM((M, KS), jnp.int32),
            pltpu.VMEM((2, K, F // 2), jnp.float32),
            pltpu.VMEM((2, K, F // 2), jnp.float32),
            pltpu.VMEM((2, K, 2 * H), jnp.float32),
            pltpu.VMEM((2, K, 2 * H), jnp.float32),
            pltpu.VMEM_SHARED((NP, F // 2), jnp.float32),
            pltpu.SemaphoreType.DMA,
            pltpu.SemaphoreType.DMA,
            pltpu.SemaphoreType.DMA,
            pltpu.SemaphoreType.DMA,
            pltpu.SemaphoreType.DMA,
            pltpu.SemaphoreType.DMA,
        ],
        compiler_params=pltpu.CompilerParams(use_tc_tiling_on_sc=False),
    )
    return f(hlo, hhi, ex, rden, src2, dst2, z64)


def _pool_body(plo0, phi0, plo1, phi1, b_r, batch_r, fw1_r, fb1_r, fw2_r,
               fb2_r, y_r):
    h3 = jnp.concatenate([plo0[...] + plo1[...], phi0[...] + phi1[...]],
                         axis=1)
    h3 = jnp.maximum(h3 + b_r[...], 0.0)
    rows = lax.broadcasted_iota(jnp.int32, (NP, 1), 0)
    h3 = jnp.where(rows < N, h3, 0.0)
    gids = lax.broadcasted_iota(jnp.int32, (1, G), 1)
    onehot = (batch_r[...] == gids).astype(jnp.float32)
    sums = lax.dot_general(onehot, h3, (((0,), (0,)), ((), ())),
                           precision=_P)
    cnt = jnp.sum(onehot, axis=0)[:, None]
    g = sums / jnp.maximum(cnt, 1.0)
    dp = lax.Precision.DEFAULT
    z = jnp.maximum(jnp.dot(g, fw1_r[...], precision=dp) + fb1_r[...], 0.0)
    y_r[...] = jnp.dot(z, fw2_r[...], precision=dp) + fb2_r[...]


def _pool(prev, b3, batchp, fw1, fb1, fw2, fb2):
    full = lambda s: pl.BlockSpec(s, lambda: (0, 0))
    return pl.pallas_call(
        _pool_body,
        in_specs=[full((NP, F // 2))] * 4 + [full((1, F)),
                  full((NP, 1)), full((F, 10)), full((1, 10)),
                  full((10, 1)), full((1, 1))],
        out_specs=full((G, 1)),
        out_shape=jax.ShapeDtypeStruct((G, 1), jnp.float32),
    )(*prev, b3, batchp, fw1, fb1, fw2, fb2)


def _dup_table(a):
    blk = a[:, :, None] * jnp.eye(H, dtype=a.dtype)[:, None, :]
    A = blk.reshape(F, H)
    return jnp.concatenate([A, A], axis=1)


def kernel(x, edge_index, batch, W1, a_s1, a_d1, b1, W2, a_s2, a_d2, b2,
           W3, a_s3, a_d3, b3, fw1, fb1, fw2, fb2):
    i32 = jnp.int32
    loop = jnp.arange(N, dtype=i32)
    pad = E_PAD - E1
    src = jnp.concatenate([edge_index[0].astype(i32), loop,
                           jnp.zeros((pad,), i32)])
    dst = jnp.concatenate([edge_index[1].astype(i32), loop,
                           jnp.full((pad,), N, i32)])
    src2 = src.reshape(E_PAD // KS, KS)
    dst2 = dst.reshape(E_PAD // KS, KS)

    xp = jnp.concatenate([x, jnp.zeros((NP - N, F), jnp.float32)])
    z16 = jnp.zeros((NP, 2 * H), jnp.float32)
    z64 = jnp.zeros((NP, F // 2), jnp.float32)
    batchp = jnp.concatenate([batch.astype(i32),
                              jnp.full((NP - N,), G, i32)]).reshape(NP, 1)

    prev = (xp,)
    bias_prev = jnp.zeros((1, F), jnp.float32)
    for li, (W, a_s, a_d, b) in enumerate(
            [(W1, a_s1, a_d1, b1), (W2, a_s2, a_d2, b2),
             (W3, a_s3, a_d3, b3)]):
        hlo, hhi, ts, td = _tc_layer(prev, bias_prev, W,
                                     _dup_table(a_s), _dup_table(a_d),
                                     li == 0)
        ex, dena, denb = _sc_a(ts, td, src2, dst2, z16)
        rden = _rden(dena, denb)
        prev = _sc_b(hlo, hhi, ex, rden, src2, dst2, z64)
        bias_prev = b.reshape(1, F)

    return _pool(prev, bias_prev, batchp, fw1, fb1.reshape(1, 10),
                 fw2, fb2.reshape(1, 1))
